# baseline (speedup 1.0000x reference)
import jax
import jax.numpy as jnp
from jax import lax
from jax.experimental import pallas as pl
from jax.experimental.pallas import tpu as pltpu
from jax.experimental.pallas import tpu_sc as plsc

N = 10000
E = 320000
F_IN = 128
F_OUT = 128
H = 6
ALPHA_COEFF = 0.2
NEG_SLOPE = 0.2

L = 16
NC = 2
NS = 16
NW = NC * NS
B1 = 64
B2 = 16
N_PAD = 10240
E1 = E + N
NBATCH1 = -(-E1 // (NW * B1))
EPW = NBATCH1 * B1
NBATCH2 = EPW // B2
GK = 54
NGRP = NBATCH2 // GK
GB = GK * B2
E_PAD = EPW * NW
GK1 = 18
NGRP1 = NBATCH1 // GK1
GB1 = GK1 * B1
ROWS_PER_SUB = N_PAD // NS
DRAIN1 = ROWS_PER_SUB // B1
DRAIN2 = ROWS_PER_SUB // B2

_SELU_SCALE = 1.0507009873554805
_SELU_ALPHA = 1.6732632423543772

_SC_PARAMS = pltpu.CompilerParams(use_tc_tiling_on_sc=False,
                                  needs_layout_passes=False)


_R1 = 640


def _stage1_body(data_ref, w_ref, aws_ref, awd_ref, x_ref, as_ref, ad_ref,
                 m_ref):
    i = pl.program_id(0)
    xb = jnp.dot(data_ref[...], w_ref[...],
                 preferred_element_type=jnp.float32,
                 precision=lax.Precision.HIGHEST)
    x_ref[...] = xb
    asb = jnp.dot(xb, aws_ref[...], preferred_element_type=jnp.float32,
                  precision=lax.Precision.HIGHEST)
    adb = jnp.dot(xb, awd_ref[...], preferred_element_type=jnp.float32,
                  precision=lax.Precision.HIGHEST)
    as_ref[...] = asb
    ad_ref[...] = adb
    ms = jnp.max(asb, axis=0, keepdims=True)
    md = jnp.max(adb, axis=0, keepdims=True)

    @pl.when(i == 0)
    def _():
        m_ref[0:1, :] = ms
        m_ref[1:2, :] = md

    @pl.when(i > 0)
    def _():
        m_ref[0:1, :] = jnp.maximum(m_ref[0:1, :], ms)
        m_ref[1:2, :] = jnp.maximum(m_ref[1:2, :], md)

    @pl.when(i == (N_PAD // _R1) - 1)
    def _():
        s = m_ref[0:1, :] + m_ref[1:2, :]
        lk = jnp.where(s >= 0.0, s, NEG_SLOPE * s)
        lane = lax.broadcasted_iota(jnp.int32, (1, L), 1)
        m_ref[2:3, :] = jnp.where(lane < H, lk, 0.0)


def _stage1(data_p, w, aws, awd):
    nblk = N_PAD // _R1
    return pl.pallas_call(
        _stage1_body,
        grid=(nblk,),
        in_specs=[
            pl.BlockSpec((_R1, F_IN), lambda i: (i, 0)),
            pl.BlockSpec((F_IN, H * F_OUT), lambda i: (0, 0)),
            pl.BlockSpec((H * F_OUT, L), lambda i: (0, 0)),
            pl.BlockSpec((H * F_OUT, L), lambda i: (0, 0)),
        ],
        out_specs=[
            pl.BlockSpec((_R1, H * F_OUT), lambda i: (i, 0)),
            pl.BlockSpec((_R1, L), lambda i: (i, 0)),
            pl.BlockSpec((_R1, L), lambda i: (i, 0)),
            pl.BlockSpec((8, L), lambda i: (0, 0)),
        ],
        out_shape=[
            jax.ShapeDtypeStruct((N_PAD, H * F_OUT), jnp.float32),
            jax.ShapeDtypeStruct((N_PAD, L), jnp.float32),
            jax.ShapeDtypeStruct((N_PAD, L), jnp.float32),
            jax.ShapeDtypeStruct((8, L), jnp.float32),
        ],
    )(data_p, w, aws, awd)


def _pass1_body(asrc_hbm, adst_hbm, src3_hbm, dst3_hbm, m_hbm,
                ex_hbm, den0_hbm, den1_hbm,
                sg_v, dg_v, asA, asB, adA, adB, ex_v, m_v, den_sh,
                semA, semB):
    cid = lax.axis_index("c")
    sid = lax.axis_index("s")
    wid = cid * NS + sid
    pltpu.sync_copy(m_hbm, m_v)
    mv = m_v[...]

    def _z(b, _):
        ex_v[b] = jnp.zeros((L,), jnp.float32)
        return 0
    lax.fori_loop(0, B1, _z, 0)
    for k in range(DRAIN1):
        pltpu.sync_copy(ex_v, den_sh.at[pl.ds(sid * ROWS_PER_SUB + k * B1, B1)])
    plsc.subcore_barrier()

    def issue(slot, as_v, ad_v, sem):
        pltpu.async_copy(asrc_hbm.at[sg_v.at[slot]], as_v, sem)
        pltpu.async_copy(adst_hbm.at[dg_v.at[slot]], ad_v, sem)

    def wait(slot, as_v, ad_v, sem):
        pltpu.make_async_copy(asrc_hbm.at[sg_v.at[slot]], as_v, sem).wait()
        pltpu.make_async_copy(adst_hbm.at[dg_v.at[slot]], ad_v, sem).wait()

    def compute(gbase, slot, as_v, ad_v):
        base = gbase + slot * B1

        @plsc.parallel_loop(0, B1, 1, unroll=2)
        def _edge(b):
            s = as_v[b] + ad_v[b]
            lk = jnp.where(s >= 0.0, s, NEG_SLOPE * s)
            ex_v[b] = jnp.exp(lk - mv)
        pltpu.sync_copy(ex_v, den_sh.at[dg_v.at[slot]], add=True)
        pltpu.sync_copy(ex_v, ex_hbm.at[pl.ds(base, B1)])

    def _group(g, _):
        grow = wid * NGRP1 + g
        gbase = wid * EPW + g * GB1
        pltpu.sync_copy(src3_hbm.at[grow], sg_v)
        pltpu.sync_copy(dst3_hbm.at[grow], dg_v)
        issue(0, asA, adA, semA)

        def _pair(k, _):
            sA = 2 * k
            sB = sA + 1
            issue(sB, asB, adB, semB)
            wait(sA, asA, adA, semA)
            compute(gbase, sA, asA, adA)
            issue(sA + 2, asA, adA, semA)
            wait(sB, asB, adB, semB)
            compute(gbase, sB, asB, adB)
            return 0
        lax.fori_loop(0, GK1 // 2 - 1, _pair, 0)
        issue(GK1 - 1, asB, adB, semB)
        wait(GK1 - 2, asA, adA, semA)
        compute(gbase, GK1 - 2, asA, adA)
        wait(GK1 - 1, asB, adB, semB)
        compute(gbase, GK1 - 1, asB, adB)
        return 0
    lax.fori_loop(0, NGRP1, _group, 0)
    plsc.subcore_barrier()

    for k in range(DRAIN1):
        r0 = sid * ROWS_PER_SUB + k * B1
        pltpu.sync_copy(den_sh.at[pl.ds(r0, B1)], asA)

        @pl.when(cid == 0)
        def _():
            pltpu.sync_copy(asA, den0_hbm.at[pl.ds(r0, B1)])

        @pl.when(cid == 1)
        def _():
            pltpu.sync_copy(asA, den1_hbm.at[pl.ds(r0, B1)])


def _pass1(asrc_p, adst_p, src3a, dst3a, mvec):
    mesh = plsc.VectorSubcoreMesh(core_axis_name="c", subcore_axis_name="s")
    kfn = pl.kernel(
        _pass1_body,
        out_type=[
            jax.ShapeDtypeStruct((E_PAD, L), jnp.float32),
            jax.ShapeDtypeStruct((N_PAD, L), jnp.float32),
            jax.ShapeDtypeStruct((N_PAD, L), jnp.float32),
        ],
        mesh=mesh,
        scratch_types=[
            pltpu.VMEM((GK1, B1), jnp.int32),
            pltpu.VMEM((GK1, B1), jnp.int32),
            pltpu.VMEM((B1, L), jnp.float32),
            pltpu.VMEM((B1, L), jnp.float32),
            pltpu.VMEM((B1, L), jnp.float32),
            pltpu.VMEM((B1, L), jnp.float32),
            pltpu.VMEM((B1, L), jnp.float32),
            pltpu.VMEM((L,), jnp.float32),
            pltpu.VMEM_SHARED((N_PAD, L), jnp.float32),
            pltpu.SemaphoreType.DMA,
            pltpu.SemaphoreType.DMA,
        ],
        compiler_params=_SC_PARAMS,
    )
    return kfn(asrc_p, adst_p, src3a, dst3a, mvec)


_RD = 1024


def _densum_body(d0_ref, d1_ref, r_ref):
    s = d0_ref[...] + d1_ref[...] + 1e-16
    lane = lax.broadcasted_iota(jnp.int32, (1, L), 1)
    r_ref[...] = jnp.where(lane < H, 1.0 / s, 0.0)


def _densum(d0, d1):
    return pl.pallas_call(
        _densum_body,
        grid=(N_PAD // _RD,),
        in_specs=[
            pl.BlockSpec((_RD, L), lambda i: (i, 0)),
            pl.BlockSpec((_RD, L), lambda i: (i, 0)),
        ],
        out_specs=pl.BlockSpec((_RD, L), lambda i: (i, 0)),
        out_shape=jax.ShapeDtypeStruct((N_PAD, L), jnp.float32),
    )(d0, d1)


def _pass2_body(x_hbm, ex_hbm, src3_hbm, dst3_hbm, rinv_hbm, coord_hbm,
                feat0_hbm, feat1_hbm, coordo0_hbm, coordo1_hbm,
                sg_v, dg_v, xA, xB, exA, exB, rA, rB, cA, cB,
                cwA, cwB, msgA, msgB, dscA, dscB, feat_sh, coord_sh,
                semA, semB, semSA, semSB):
    cid = lax.axis_index("c")
    sid = lax.axis_index("s")
    wid = cid * NS + sid

    def issue(gbase, slot, x_v, ex_v, r_v, c_v, sem):
        base = gbase + slot * B2
        pltpu.async_copy(x_hbm.at[sg_v.at[slot]], x_v, sem)
        pltpu.async_copy(ex_hbm.at[pl.ds(base, B2)], ex_v, sem)
        pltpu.async_copy(rinv_hbm.at[dg_v.at[slot]], r_v, sem)
        pltpu.async_copy(coord_hbm.at[sg_v.at[slot]], c_v, sem)

    def wait(gbase, slot, x_v, ex_v, r_v, c_v, sem):
        base = gbase + slot * B2
        pltpu.make_async_copy(x_hbm.at[sg_v.at[slot]], x_v, sem).wait()
        pltpu.make_async_copy(ex_hbm.at[pl.ds(base, B2)], ex_v, sem).wait()
        pltpu.make_async_copy(rinv_hbm.at[dg_v.at[slot]], r_v, sem).wait()
        pltpu.make_async_copy(coord_hbm.at[sg_v.at[slot]], c_v, sem).wait()

    def compute(slot, x_v, ex_v, r_v, c_v, msg_v, cw_v, dsc_v, semS):
        pltpu.make_async_copy(msg_v, feat_sh.at[dsc_v], semS).wait()
        pltpu.make_async_copy(cw_v, coord_sh.at[dsc_v], semS).wait()

        @plsc.parallel_loop(0, B2, 1, unroll=2)
        def _edge(b):
            al = ex_v[b] * r_v[b]
            asum = jnp.sum(al)
            cw_v[b] = c_v[b] * asum
            acc = [None] * (F_OUT // L)
            for h in range(H):
                ah = al[h]
                for q in range(F_OUT // (2 * L)):
                    v = x_v[b, pl.ds(h * F_OUT + 2 * L * q, 2 * L)]
                    e, o = plsc.unpack(v, format=plsc.PackFormat.INTERLEAVED)
                    if h == 0:
                        acc[2 * q] = ah * e
                        acc[2 * q + 1] = ah * o
                    else:
                        acc[2 * q] = acc[2 * q] + ah * e
                        acc[2 * q + 1] = acc[2 * q + 1] + ah * o
            for j in range(F_OUT // L):
                msg_v[b, pl.ds(j * L, L)] = acc[j]
        dsc_v[...] = dg_v[slot]
        pltpu.async_copy(msg_v, feat_sh.at[dsc_v], semS, add=True)
        pltpu.async_copy(cw_v, coord_sh.at[dsc_v], semS, add=True)

    def _zf(b, _):
        for j in range(F_OUT // L):
            msgA[b, pl.ds(j * L, L)] = jnp.zeros((L,), jnp.float32)
            msgB[b, pl.ds(j * L, L)] = jnp.zeros((L,), jnp.float32)
        cwA[b] = jnp.zeros((L,), jnp.float32)
        cwB[b] = jnp.zeros((L,), jnp.float32)
        return 0
    lax.fori_loop(0, B2, _zf, 0)
    for k in range(DRAIN2):
        r0 = sid * ROWS_PER_SUB + k * B2
        pltpu.sync_copy(msgA, feat_sh.at[pl.ds(r0, B2)])
        pltpu.sync_copy(cwA, coord_sh.at[pl.ds(r0, B2)])
    plsc.subcore_barrier()

    dscA[...] = lax.broadcasted_iota(jnp.int32, (L,), 0)
    dscB[...] = lax.broadcasted_iota(jnp.int32, (L,), 0)
    pltpu.async_copy(msgA, feat_sh.at[dscA], semSA, add=True)
    pltpu.async_copy(cwA, coord_sh.at[dscA], semSA, add=True)
    pltpu.async_copy(msgB, feat_sh.at[dscB], semSB, add=True)
    pltpu.async_copy(cwB, coord_sh.at[dscB], semSB, add=True)

    def _group(g, _):
        grow = wid * NGRP + g
        gbase = wid * EPW + g * GB
        pltpu.sync_copy(src3_hbm.at[grow], sg_v)
        pltpu.sync_copy(dst3_hbm.at[grow], dg_v)
        issue(gbase, 0, xA, exA, rA, cA, semA)

        def _pair(k, _):
            sA = 2 * k
            sB = sA + 1
            issue(gbase, sB, xB, exB, rB, cB, semB)
            wait(gbase, sA, xA, exA, rA, cA, semA)
            compute(sA, xA, exA, rA, cA, msgA, cwA, dscA, semSA)
            issue(gbase, sA + 2, xA, exA, rA, cA, semA)
            wait(gbase, sB, xB, exB, rB, cB, semB)
            compute(sB, xB, exB, rB, cB, msgB, cwB, dscB, semSB)
            return 0
        lax.fori_loop(0, GK // 2 - 1, _pair, 0)
        issue(gbase, GK - 1, xB, exB, rB, cB, semB)
        wait(gbase, GK - 2, xA, exA, rA, cA, semA)
        compute(GK - 2, xA, exA, rA, cA, msgA, cwA, dscA, semSA)
        wait(gbase, GK - 1, xB, exB, rB, cB, semB)
        compute(GK - 1, xB, exB, rB, cB, msgB, cwB, dscB, semSB)
        return 0
    lax.fori_loop(0, NGRP, _group, 0)
    pltpu.make_async_copy(msgA, feat_sh.at[dscA], semSA).wait()
    pltpu.make_async_copy(cwA, coord_sh.at[dscA], semSA).wait()
    pltpu.make_async_copy(msgB, feat_sh.at[dscB], semSB).wait()
    pltpu.make_async_copy(cwB, coord_sh.at[dscB], semSB).wait()
    plsc.subcore_barrier()

    for k in range(DRAIN2):
        r0 = sid * ROWS_PER_SUB + k * B2
        pltpu.sync_copy(feat_sh.at[pl.ds(r0, B2)], msgA)
        pltpu.sync_copy(coord_sh.at[pl.ds(r0, B2)], cwA)

        @pl.when(cid == 0)
        def _():
            pltpu.sync_copy(msgA, feat0_hbm.at[pl.ds(r0, B2)])
            pltpu.sync_copy(cwA, coordo0_hbm.at[pl.ds(r0, B2)])

        @pl.when(cid == 1)
        def _():
            pltpu.sync_copy(msgA, feat1_hbm.at[pl.ds(r0, B2)])
            pltpu.sync_copy(cwA, coordo1_hbm.at[pl.ds(r0, B2)])


def _pass2(x_p, ex, src3, dst3, rinv, coord_p):
    mesh = plsc.VectorSubcoreMesh(core_axis_name="c", subcore_axis_name="s")
    kfn = pl.kernel(
        _pass2_body,
        out_type=[
            jax.ShapeDtypeStruct((N_PAD, F_OUT), jnp.float32),
            jax.ShapeDtypeStruct((N_PAD, F_OUT), jnp.float32),
            jax.ShapeDtypeStruct((N_PAD, L), jnp.float32),
            jax.ShapeDtypeStruct((N_PAD, L), jnp.float32),
        ],
        mesh=mesh,
        scratch_types=[
            pltpu.VMEM((GK, B2), jnp.int32),
            pltpu.VMEM((GK, B2), jnp.int32),
            pltpu.VMEM((B2, H * F_OUT), jnp.bfloat16),
            pltpu.VMEM((B2, H * F_OUT), jnp.bfloat16),
            pltpu.VMEM((B2, L), jnp.float32),
            pltpu.VMEM((B2, L), jnp.float32),
            pltpu.VMEM((B2, L), jnp.float32),
            pltpu.VMEM((B2, L), jnp.float32),
            pltpu.VMEM((B2, L), jnp.float32),
            pltpu.VMEM((B2, L), jnp.float32),
            pltpu.VMEM((B2, L), jnp.float32),
            pltpu.VMEM((B2, L), jnp.float32),
            pltpu.VMEM((B2, F_OUT), jnp.float32),
            pltpu.VMEM((B2, F_OUT), jnp.float32),
            pltpu.VMEM((L,), jnp.int32),
            pltpu.VMEM((L,), jnp.int32),
            pltpu.VMEM_SHARED((N_PAD, F_OUT), jnp.float32),
            pltpu.VMEM_SHARED((N_PAD, L), jnp.float32),
            pltpu.SemaphoreType.DMA,
            pltpu.SemaphoreType.DMA,
            pltpu.SemaphoreType.DMA,
            pltpu.SemaphoreType.DMA,
        ],
        compiler_params=_SC_PARAMS,
    )
    return kfn(x_p, ex, src3, dst3, rinv, coord_p)


_R3 = 1000


def _stage3_body(f0_ref, f1_ref, c0_ref, c1_ref, data_ref, bias_ref,
                 coord_ref, feat_ref):
    fsum = (f0_ref[...] + f1_ref[...]) * (1.0 / H)
    xv = fsum + bias_ref[...]
    feat_ref[...] = _SELU_SCALE * jnp.where(
        xv > 0.0, xv, _SELU_ALPHA * (jnp.exp(xv) - 1.0))
    csum = (c0_ref[...] + c1_ref[...]) * (ALPHA_COEFF / H)
    d = data_ref[...]
    col0 = d[:, 0:1]
    col1 = d[:, 1:2]
    cc0 = csum[:, 0:1]
    cc1 = csum[:, 1:2]
    cc0 = jnp.where(col0 == 1.0, 1.0, cc0)
    cc0 = jnp.where(col0 == 0.0, 0.0, cc0)
    cc1 = jnp.where(col1 == 0.0, 0.0, cc1)
    cc1 = jnp.where(col1 == 1.0, 1.0, cc1)
    coord_ref[...] = jnp.concatenate([cc0, cc1], axis=1)


def _stage3(f0, f1, c0, c1, data, bias2d):
    nblk = N // _R3
    return pl.pallas_call(
        _stage3_body,
        grid=(nblk,),
        in_specs=[
            pl.BlockSpec((_R3, F_OUT), lambda i: (i, 0)),
            pl.BlockSpec((_R3, F_OUT), lambda i: (i, 0)),
            pl.BlockSpec((_R3, L), lambda i: (i, 0)),
            pl.BlockSpec((_R3, L), lambda i: (i, 0)),
            pl.BlockSpec((_R3, F_IN), lambda i: (i, 0)),
            pl.BlockSpec((1, F_OUT), lambda i: (0, 0)),
        ],
        out_specs=[
            pl.BlockSpec((_R3, 2), lambda i: (i, 0)),
            pl.BlockSpec((_R3, F_OUT), lambda i: (i, 0)),
        ],
        out_shape=[
            jax.ShapeDtypeStruct((N, 2), jnp.float32),
            jax.ShapeDtypeStruct((N, F_OUT), jnp.float32),
        ],
    )(f0, f1, c0, c1, data, bias2d)


@jax.jit
def kernel(data, edge_idx, W, att_src, att_dst, bias):
    data = data.astype(jnp.float32)
    W = W.astype(jnp.float32)

    rows = jnp.arange(H * F_OUT, dtype=jnp.int32)
    cols = jnp.repeat(jnp.arange(H, dtype=jnp.int32), F_OUT)
    aws = jnp.zeros((H * F_OUT, L), jnp.float32).at[rows, cols].set(
        att_src.reshape(-1).astype(jnp.float32))
    awd = jnp.zeros((H * F_OUT, L), jnp.float32).at[rows, cols].set(
        att_dst.reshape(-1).astype(jnp.float32))

    data_p = jnp.concatenate(
        [data, jnp.zeros((N_PAD - N, F_IN), jnp.float32)], axis=0)
    x_p, asrc_p, adst_p, mrows = _stage1(data_p, W, aws, awd)
    mvec = mrows[2]

    coord_p = jnp.zeros((N_PAD, L), jnp.float32).at[:N, 0:2].set(data[:, 0:2])

    loop = jnp.arange(N, dtype=jnp.int32)
    padi = jnp.full((E_PAD - E1,), N, dtype=jnp.int32)
    src_p = jnp.concatenate([edge_idx[0].astype(jnp.int32), loop, padi])
    dst_p = jnp.concatenate([edge_idx[1].astype(jnp.int32), loop, padi])
    src3 = src_p.reshape(NW * NGRP, GK, B2)
    dst3 = dst_p.reshape(NW * NGRP, GK, B2)
    src3a = src_p.reshape(NW * NGRP1, GK1, B1)
    dst3a = dst_p.reshape(NW * NGRP1, GK1, B1)

    ex, den0, den1 = _pass1(asrc_p, adst_p, src3a, dst3a, mvec)
    rinv = _densum(den0, den1)
    x_bf = (x_p.reshape(N_PAD, H * F_OUT // (2 * L), 2, L)
            .transpose(0, 1, 3, 2)
            .reshape(N_PAD, H * F_OUT).astype(jnp.bfloat16))
    featp0, featp1, coordp0, coordp1 = _pass2(x_bf, ex, src3, dst3, rinv,
                                              coord_p)

    out_coord, feat = _stage3(featp0, featp1, coordp0, coordp1, data,
                              bias.astype(jnp.float32).reshape(1, F_OUT))
    return out_coord, feat


# speedup vs baseline: 1.1242x; 1.1242x over previous
import jax
import jax.numpy as jnp
from jax import lax
from jax.experimental import pallas as pl
from jax.experimental.pallas import tpu as pltpu
from jax.experimental.pallas import tpu_sc as plsc

N = 10000
E = 320000
F_IN = 128
F_OUT = 128
H = 6
ALPHA_COEFF = 0.2
NEG_SLOPE = 0.2

L = 16
NC = 2
NS = 16
NW = NC * NS
B1 = 64
B2 = 16
N_PAD = 10240
E1 = E + N
NBATCH1 = -(-E1 // (NW * B1))
EPW = NBATCH1 * B1
NBATCH2 = EPW // B2
GK = 54
NGRP = NBATCH2 // GK
GB = GK * B2
E_PAD = EPW * NW
GK1 = 18
NGRP1 = NBATCH1 // GK1
GB1 = GK1 * B1
ROWS_PER_SUB = N_PAD // NS
DRAIN1 = ROWS_PER_SUB // B1
DRAIN2 = ROWS_PER_SUB // B2

_SELU_SCALE = 1.0507009873554805
_SELU_ALPHA = 1.6732632423543772

_SC_PARAMS = pltpu.CompilerParams(use_tc_tiling_on_sc=False,
                                  needs_layout_passes=False)


_R1 = 640


def _stage1_body(data_ref, w_ref, aws_ref, awd_ref, x_ref, as_ref, ad_ref,
                 m_ref):
    i = pl.program_id(0)
    xb = jnp.dot(data_ref[...], w_ref[...],
                 preferred_element_type=jnp.float32,
                 precision=lax.Precision.HIGHEST)
    x_ref[...] = xb.astype(jnp.bfloat16)
    asb = jnp.dot(xb, aws_ref[...], preferred_element_type=jnp.float32,
                  precision=lax.Precision.HIGHEST)
    adb = jnp.dot(xb, awd_ref[...], preferred_element_type=jnp.float32,
                  precision=lax.Precision.HIGHEST)
    as_ref[...] = asb
    ad_ref[...] = adb
    ms = jnp.max(asb, axis=0, keepdims=True)
    md = jnp.max(adb, axis=0, keepdims=True)

    @pl.when(i == 0)
    def _():
        m_ref[0:1, :] = ms
        m_ref[1:2, :] = md

    @pl.when(i > 0)
    def _():
        m_ref[0:1, :] = jnp.maximum(m_ref[0:1, :], ms)
        m_ref[1:2, :] = jnp.maximum(m_ref[1:2, :], md)

    @pl.when(i == (N_PAD // _R1) - 1)
    def _():
        s = m_ref[0:1, :] + m_ref[1:2, :]
        lk = jnp.where(s >= 0.0, s, NEG_SLOPE * s)
        lane = lax.broadcasted_iota(jnp.int32, (1, L), 1)
        m_ref[2:3, :] = jnp.where(lane < H, lk, 0.0)


def _stage1(data_p, w, aws, awd):
    nblk = N_PAD // _R1
    return pl.pallas_call(
        _stage1_body,
        grid=(nblk,),
        in_specs=[
            pl.BlockSpec((_R1, F_IN), lambda i: (i, 0)),
            pl.BlockSpec((F_IN, H * F_OUT), lambda i: (0, 0)),
            pl.BlockSpec((H * F_OUT, L), lambda i: (0, 0)),
            pl.BlockSpec((H * F_OUT, L), lambda i: (0, 0)),
        ],
        out_specs=[
            pl.BlockSpec((_R1, H * F_OUT), lambda i: (i, 0)),
            pl.BlockSpec((_R1, L), lambda i: (i, 0)),
            pl.BlockSpec((_R1, L), lambda i: (i, 0)),
            pl.BlockSpec((8, L), lambda i: (0, 0)),
        ],
        out_shape=[
            jax.ShapeDtypeStruct((N_PAD, H * F_OUT), jnp.bfloat16),
            jax.ShapeDtypeStruct((N_PAD, L), jnp.float32),
            jax.ShapeDtypeStruct((N_PAD, L), jnp.float32),
            jax.ShapeDtypeStruct((8, L), jnp.float32),
        ],
    )(data_p, w, aws, awd)


def _pass1_body(asrc_hbm, adst_hbm, src3_hbm, dst3_hbm, m_hbm,
                ex_hbm, den0_hbm, den1_hbm,
                sg_v, dg_v, asA, asB, adA, adB, ex_v, m_v, den_sh,
                semA, semB):
    cid = lax.axis_index("c")
    sid = lax.axis_index("s")
    wid = cid * NS + sid
    pltpu.sync_copy(m_hbm, m_v)
    mv = m_v[...]

    def _z(b, _):
        ex_v[b] = jnp.zeros((L,), jnp.float32)
        return 0
    lax.fori_loop(0, B1, _z, 0)
    for k in range(DRAIN1):
        pltpu.sync_copy(ex_v, den_sh.at[pl.ds(sid * ROWS_PER_SUB + k * B1, B1)])
    plsc.subcore_barrier()

    def issue(slot, as_v, ad_v, sem):
        pltpu.async_copy(asrc_hbm.at[sg_v.at[slot]], as_v, sem)
        pltpu.async_copy(adst_hbm.at[dg_v.at[slot]], ad_v, sem)

    def wait(slot, as_v, ad_v, sem):
        pltpu.make_async_copy(asrc_hbm.at[sg_v.at[slot]], as_v, sem).wait()
        pltpu.make_async_copy(adst_hbm.at[dg_v.at[slot]], ad_v, sem).wait()

    def compute(gbase, slot, as_v, ad_v):
        base = gbase + slot * B1

        @plsc.parallel_loop(0, B1, 1, unroll=2)
        def _edge(b):
            s = as_v[b] + ad_v[b]
            lk = jnp.where(s >= 0.0, s, NEG_SLOPE * s)
            ex_v[b] = jnp.exp(lk - mv)
        pltpu.sync_copy(ex_v, den_sh.at[dg_v.at[slot]], add=True)
        pltpu.sync_copy(ex_v, ex_hbm.at[pl.ds(base, B1)])

    def _group(g, _):
        grow = wid * NGRP1 + g
        gbase = wid * EPW + g * GB1
        pltpu.sync_copy(src3_hbm.at[grow], sg_v)
        pltpu.sync_copy(dst3_hbm.at[grow], dg_v)
        issue(0, asA, adA, semA)

        def _pair(k, _):
            sA = 2 * k
            sB = sA + 1
            issue(sB, asB, adB, semB)
            wait(sA, asA, adA, semA)
            compute(gbase, sA, asA, adA)
            issue(sA + 2, asA, adA, semA)
            wait(sB, asB, adB, semB)
            compute(gbase, sB, asB, adB)
            return 0
        lax.fori_loop(0, GK1 // 2 - 1, _pair, 0)
        issue(GK1 - 1, asB, adB, semB)
        wait(GK1 - 2, asA, adA, semA)
        compute(gbase, GK1 - 2, asA, adA)
        wait(GK1 - 1, asB, adB, semB)
        compute(gbase, GK1 - 1, asB, adB)
        return 0
    lax.fori_loop(0, NGRP1, _group, 0)
    plsc.subcore_barrier()

    for k in range(DRAIN1):
        r0 = sid * ROWS_PER_SUB + k * B1
        pltpu.sync_copy(den_sh.at[pl.ds(r0, B1)], asA)

        @pl.when(cid == 0)
        def _():
            pltpu.sync_copy(asA, den0_hbm.at[pl.ds(r0, B1)])

        @pl.when(cid == 1)
        def _():
            pltpu.sync_copy(asA, den1_hbm.at[pl.ds(r0, B1)])


def _pass1(asrc_p, adst_p, src3a, dst3a, mvec):
    mesh = plsc.VectorSubcoreMesh(core_axis_name="c", subcore_axis_name="s")
    kfn = pl.kernel(
        _pass1_body,
        out_type=[
            jax.ShapeDtypeStruct((E_PAD, L), jnp.float32),
            jax.ShapeDtypeStruct((N_PAD, L), jnp.float32),
            jax.ShapeDtypeStruct((N_PAD, L), jnp.float32),
        ],
        mesh=mesh,
        scratch_types=[
            pltpu.VMEM((GK1, B1), jnp.int32),
            pltpu.VMEM((GK1, B1), jnp.int32),
            pltpu.VMEM((B1, L), jnp.float32),
            pltpu.VMEM((B1, L), jnp.float32),
            pltpu.VMEM((B1, L), jnp.float32),
            pltpu.VMEM((B1, L), jnp.float32),
            pltpu.VMEM((B1, L), jnp.float32),
            pltpu.VMEM((L,), jnp.float32),
            pltpu.VMEM_SHARED((N_PAD, L), jnp.float32),
            pltpu.SemaphoreType.DMA,
            pltpu.SemaphoreType.DMA,
        ],
        compiler_params=_SC_PARAMS,
    )
    return kfn(asrc_p, adst_p, src3a, dst3a, mvec)


_RD = 1024


def _densum_body(d0_ref, d1_ref, r_ref):
    s = d0_ref[...] + d1_ref[...] + 1e-16
    lane = lax.broadcasted_iota(jnp.int32, (1, L), 1)
    r_ref[...] = jnp.where(lane < H, 1.0 / s, 0.0)


def _densum(d0, d1):
    return pl.pallas_call(
        _densum_body,
        grid=(N_PAD // _RD,),
        in_specs=[
            pl.BlockSpec((_RD, L), lambda i: (i, 0)),
            pl.BlockSpec((_RD, L), lambda i: (i, 0)),
        ],
        out_specs=pl.BlockSpec((_RD, L), lambda i: (i, 0)),
        out_shape=jax.ShapeDtypeStruct((N_PAD, L), jnp.float32),
    )(d0, d1)


def _pass2_body(x_hbm, ex_hbm, src3_hbm, dst3_hbm, rinv_hbm, coord_hbm,
                feat0_hbm, feat1_hbm, coordo0_hbm, coordo1_hbm,
                sg_v, dg_v, xA, xB, exA, exB, rA, rB, cA, cB,
                cwA, cwB, msgA, msgB, dscA, dscB, feat_sh, coord_sh,
                semA, semB, semSA, semSB):
    cid = lax.axis_index("c")
    sid = lax.axis_index("s")
    wid = cid * NS + sid

    def issue(gbase, slot, x_v, ex_v, r_v, c_v, sem):
        base = gbase + slot * B2
        pltpu.async_copy(x_hbm.at[sg_v.at[slot]], x_v, sem)
        pltpu.async_copy(ex_hbm.at[pl.ds(base, B2)], ex_v, sem)
        pltpu.async_copy(rinv_hbm.at[dg_v.at[slot]], r_v, sem)
        pltpu.async_copy(coord_hbm.at[sg_v.at[slot]], c_v, sem)

    def wait(gbase, slot, x_v, ex_v, r_v, c_v, sem):
        base = gbase + slot * B2
        pltpu.make_async_copy(x_hbm.at[sg_v.at[slot]], x_v, sem).wait()
        pltpu.make_async_copy(ex_hbm.at[pl.ds(base, B2)], ex_v, sem).wait()
        pltpu.make_async_copy(rinv_hbm.at[dg_v.at[slot]], r_v, sem).wait()
        pltpu.make_async_copy(coord_hbm.at[sg_v.at[slot]], c_v, sem).wait()

    def compute(slot, x_v, ex_v, r_v, c_v, msg_v, cw_v, dsc_v, semS):
        pltpu.make_async_copy(msg_v, feat_sh.at[dsc_v], semS).wait()
        pltpu.make_async_copy(cw_v, coord_sh.at[dsc_v], semS).wait()

        @plsc.parallel_loop(0, B2, 1, unroll=2)
        def _edge(b):
            al = ex_v[b] * r_v[b]
            asum = jnp.sum(al)
            cw_v[b] = c_v[b] * asum
            acc = [None] * (F_OUT // L)
            for h in range(H):
                ah = al[h]
                for q in range(F_OUT // (2 * L)):
                    v = x_v[b, pl.ds(h * F_OUT + 2 * L * q, 2 * L)]
                    e, o = plsc.unpack(v, format=plsc.PackFormat.INTERLEAVED)
                    if h == 0:
                        acc[2 * q] = ah * e
                        acc[2 * q + 1] = ah * o
                    else:
                        acc[2 * q] = acc[2 * q] + ah * e
                        acc[2 * q + 1] = acc[2 * q + 1] + ah * o
            for j in range(F_OUT // L):
                msg_v[b, pl.ds(j * L, L)] = acc[j]
        dsc_v[...] = dg_v[slot]
        pltpu.async_copy(msg_v, feat_sh.at[dsc_v], semS, add=True)
        pltpu.async_copy(cw_v, coord_sh.at[dsc_v], semS, add=True)

    def _zf(b, _):
        for j in range(F_OUT // L):
            msgA[b, pl.ds(j * L, L)] = jnp.zeros((L,), jnp.float32)
            msgB[b, pl.ds(j * L, L)] = jnp.zeros((L,), jnp.float32)
        cwA[b] = jnp.zeros((L,), jnp.float32)
        cwB[b] = jnp.zeros((L,), jnp.float32)
        return 0
    lax.fori_loop(0, B2, _zf, 0)
    for k in range(DRAIN2):
        r0 = sid * ROWS_PER_SUB + k * B2
        pltpu.sync_copy(msgA, feat_sh.at[pl.ds(r0, B2)])
        pltpu.sync_copy(cwA, coord_sh.at[pl.ds(r0, B2)])
    plsc.subcore_barrier()

    dscA[...] = lax.broadcasted_iota(jnp.int32, (L,), 0)
    dscB[...] = lax.broadcasted_iota(jnp.int32, (L,), 0)
    pltpu.async_copy(msgA, feat_sh.at[dscA], semSA, add=True)
    pltpu.async_copy(cwA, coord_sh.at[dscA], semSA, add=True)
    pltpu.async_copy(msgB, feat_sh.at[dscB], semSB, add=True)
    pltpu.async_copy(cwB, coord_sh.at[dscB], semSB, add=True)

    def _group(g, _):
        grow = wid * NGRP + g
        gbase = wid * EPW + g * GB
        pltpu.sync_copy(src3_hbm.at[grow], sg_v)
        pltpu.sync_copy(dst3_hbm.at[grow], dg_v)
        issue(gbase, 0, xA, exA, rA, cA, semA)

        def _pair(k, _):
            sA = 2 * k
            sB = sA + 1
            issue(gbase, sB, xB, exB, rB, cB, semB)
            wait(gbase, sA, xA, exA, rA, cA, semA)
            compute(sA, xA, exA, rA, cA, msgA, cwA, dscA, semSA)
            issue(gbase, sA + 2, xA, exA, rA, cA, semA)
            wait(gbase, sB, xB, exB, rB, cB, semB)
            compute(sB, xB, exB, rB, cB, msgB, cwB, dscB, semSB)
            return 0
        lax.fori_loop(0, GK // 2 - 1, _pair, 0)
        issue(gbase, GK - 1, xB, exB, rB, cB, semB)
        wait(gbase, GK - 2, xA, exA, rA, cA, semA)
        compute(GK - 2, xA, exA, rA, cA, msgA, cwA, dscA, semSA)
        wait(gbase, GK - 1, xB, exB, rB, cB, semB)
        compute(GK - 1, xB, exB, rB, cB, msgB, cwB, dscB, semSB)
        return 0
    lax.fori_loop(0, NGRP, _group, 0)
    pltpu.make_async_copy(msgA, feat_sh.at[dscA], semSA).wait()
    pltpu.make_async_copy(cwA, coord_sh.at[dscA], semSA).wait()
    pltpu.make_async_copy(msgB, feat_sh.at[dscB], semSB).wait()
    pltpu.make_async_copy(cwB, coord_sh.at[dscB], semSB).wait()
    plsc.subcore_barrier()

    for k in range(DRAIN2):
        r0 = sid * ROWS_PER_SUB + k * B2
        pltpu.sync_copy(feat_sh.at[pl.ds(r0, B2)], msgA)
        pltpu.sync_copy(coord_sh.at[pl.ds(r0, B2)], cwA)

        @pl.when(cid == 0)
        def _():
            pltpu.sync_copy(msgA, feat0_hbm.at[pl.ds(r0, B2)])
            pltpu.sync_copy(cwA, coordo0_hbm.at[pl.ds(r0, B2)])

        @pl.when(cid == 1)
        def _():
            pltpu.sync_copy(msgA, feat1_hbm.at[pl.ds(r0, B2)])
            pltpu.sync_copy(cwA, coordo1_hbm.at[pl.ds(r0, B2)])


def _pass2(x_p, ex, src3, dst3, rinv, coord_p):
    mesh = plsc.VectorSubcoreMesh(core_axis_name="c", subcore_axis_name="s")
    kfn = pl.kernel(
        _pass2_body,
        out_type=[
            jax.ShapeDtypeStruct((N_PAD, F_OUT), jnp.float32),
            jax.ShapeDtypeStruct((N_PAD, F_OUT), jnp.float32),
            jax.ShapeDtypeStruct((N_PAD, L), jnp.float32),
            jax.ShapeDtypeStruct((N_PAD, L), jnp.float32),
        ],
        mesh=mesh,
        scratch_types=[
            pltpu.VMEM((GK, B2), jnp.int32),
            pltpu.VMEM((GK, B2), jnp.int32),
            pltpu.VMEM((B2, H * F_OUT), jnp.bfloat16),
            pltpu.VMEM((B2, H * F_OUT), jnp.bfloat16),
            pltpu.VMEM((B2, L), jnp.float32),
            pltpu.VMEM((B2, L), jnp.float32),
            pltpu.VMEM((B2, L), jnp.float32),
            pltpu.VMEM((B2, L), jnp.float32),
            pltpu.VMEM((B2, L), jnp.float32),
            pltpu.VMEM((B2, L), jnp.float32),
            pltpu.VMEM((B2, L), jnp.float32),
            pltpu.VMEM((B2, L), jnp.float32),
            pltpu.VMEM((B2, F_OUT), jnp.float32),
            pltpu.VMEM((B2, F_OUT), jnp.float32),
            pltpu.VMEM((L,), jnp.int32),
            pltpu.VMEM((L,), jnp.int32),
            pltpu.VMEM_SHARED((N_PAD, F_OUT), jnp.float32),
            pltpu.VMEM_SHARED((N_PAD, L), jnp.float32),
            pltpu.SemaphoreType.DMA,
            pltpu.SemaphoreType.DMA,
            pltpu.SemaphoreType.DMA,
            pltpu.SemaphoreType.DMA,
        ],
        compiler_params=_SC_PARAMS,
    )
    return kfn(x_p, ex, src3, dst3, rinv, coord_p)


_R3 = 1000


def _stage3_body(f0_ref, f1_ref, c0_ref, c1_ref, data_ref, bias_ref,
                 coord_ref, feat_ref):
    fsum = (f0_ref[...] + f1_ref[...]) * (1.0 / H)
    xv = fsum + bias_ref[...]
    feat_ref[...] = _SELU_SCALE * jnp.where(
        xv > 0.0, xv, _SELU_ALPHA * (jnp.exp(xv) - 1.0))
    csum = (c0_ref[...] + c1_ref[...]) * (ALPHA_COEFF / H)
    d = data_ref[...]
    col0 = d[:, 0:1]
    col1 = d[:, 1:2]
    cc0 = csum[:, 0:1]
    cc1 = csum[:, 1:2]
    cc0 = jnp.where(col0 == 1.0, 1.0, cc0)
    cc0 = jnp.where(col0 == 0.0, 0.0, cc0)
    cc1 = jnp.where(col1 == 0.0, 0.0, cc1)
    cc1 = jnp.where(col1 == 1.0, 1.0, cc1)
    coord_ref[...] = jnp.concatenate([cc0, cc1], axis=1)


def _stage3(f0, f1, c0, c1, data, bias2d):
    nblk = N // _R3
    return pl.pallas_call(
        _stage3_body,
        grid=(nblk,),
        in_specs=[
            pl.BlockSpec((_R3, F_OUT), lambda i: (i, 0)),
            pl.BlockSpec((_R3, F_OUT), lambda i: (i, 0)),
            pl.BlockSpec((_R3, L), lambda i: (i, 0)),
            pl.BlockSpec((_R3, L), lambda i: (i, 0)),
            pl.BlockSpec((_R3, F_IN), lambda i: (i, 0)),
            pl.BlockSpec((1, F_OUT), lambda i: (0, 0)),
        ],
        out_specs=[
            pl.BlockSpec((_R3, 2), lambda i: (i, 0)),
            pl.BlockSpec((_R3, F_OUT), lambda i: (i, 0)),
        ],
        out_shape=[
            jax.ShapeDtypeStruct((N, 2), jnp.float32),
            jax.ShapeDtypeStruct((N, F_OUT), jnp.float32),
        ],
    )(f0, f1, c0, c1, data, bias2d)


@jax.jit
def kernel(data, edge_idx, W, att_src, att_dst, bias):
    data = data.astype(jnp.float32)
    W = W.astype(jnp.float32)

    qi = jnp.arange(H * F_OUT, dtype=jnp.int32)
    qb = qi // 32
    rr = qi % 32
    perm_src = qb * 32 + (rr % 2) * 16 + rr // 2
    w_perm = W[:, perm_src]

    rows = jnp.arange(H * F_OUT, dtype=jnp.int32)
    cols = jnp.repeat(jnp.arange(H, dtype=jnp.int32), F_OUT)
    aws = jnp.zeros((H * F_OUT, L), jnp.float32).at[rows, cols].set(
        att_src.reshape(-1).astype(jnp.float32)[perm_src])
    awd = jnp.zeros((H * F_OUT, L), jnp.float32).at[rows, cols].set(
        att_dst.reshape(-1).astype(jnp.float32)[perm_src])

    data_p = jnp.concatenate(
        [data, jnp.zeros((N_PAD - N, F_IN), jnp.float32)], axis=0)
    x_bf, asrc_p, adst_p, mrows = _stage1(data_p, w_perm, aws, awd)
    mvec = mrows[2]

    coord_p = jnp.zeros((N_PAD, L), jnp.float32).at[:N, 0:2].set(data[:, 0:2])

    loop = jnp.arange(N, dtype=jnp.int32)
    padi = jnp.full((E_PAD - E1,), N, dtype=jnp.int32)
    src_p = jnp.concatenate([edge_idx[0].astype(jnp.int32), loop, padi])
    dst_p = jnp.concatenate([edge_idx[1].astype(jnp.int32), loop, padi])
    src3 = src_p.reshape(NW * NGRP, GK, B2)
    dst3 = dst_p.reshape(NW * NGRP, GK, B2)
    src3a = src_p.reshape(NW * NGRP1, GK1, B1)
    dst3a = dst_p.reshape(NW * NGRP1, GK1, B1)

    ex, den0, den1 = _pass1(asrc_p, adst_p, src3a, dst3a, mvec)
    rinv = _densum(den0, den1)
    featp0, featp1, coordp0, coordp1 = _pass2(x_bf, ex, src3, dst3, rinv,
                                              coord_p)

    out_coord, feat = _stage3(featp0, featp1, coordp0, coordp1, data,
                              bias.astype(jnp.float32).reshape(1, F_OUT))
    return out_coord, feat
